# baseline (speedup 1.0000x reference)
import jax
import jax.numpy as jnp
from jax.experimental import pallas as pl
from jax.experimental.pallas import tpu as pltpu

_HF = 112
_CBLK = 16
_TOP = slice(0, _HF)
_BOT = slice(_HF, 2 * _HF)


def _dma_plan():
    plan = []
    for k in range(6):
        cs = slice(40 * k, 40 * (k + 1))
        plan.append(((slice(None), cs), (slice(None), cs)))
    cs = slice(288, 336)
    plan.append(((slice(None), cs, _BOT), (slice(None), cs, _TOP)))
    plan.append(((slice(None), cs, _TOP), (slice(None), cs, _BOT)))
    return plan


_PLAN = _dma_plan()


def _dma_body(x_hbm, o_hbm, sems):
    copies = []
    for i, (src, dst) in enumerate(_PLAN):
        cp = pltpu.make_async_copy(x_hbm.at[src], o_hbm.at[dst], sems.at[i])
        cp.start()
        copies.append(cp)
    for cp in copies:
        cp.wait()


def _rot_body(y_ref, x_ref, o_ref):
    del y_ref
    j = pl.program_id(1)

    @pl.when(j < 48 // _CBLK)
    def _():
        o_ref[:, :, _TOP, _TOP] = x_ref[:, :, _TOP, _BOT]
        o_ref[:, :, _TOP, _BOT] = x_ref[:, :, _BOT, _TOP]
        o_ref[:, :, _BOT, _TOP] = x_ref[:, :, _BOT, _BOT]
        o_ref[:, :, _BOT, _BOT] = x_ref[:, :, _TOP, _TOP]

    @pl.when(j >= 48 // _CBLK)
    def _():
        o_ref[:, :, _TOP, _TOP] = x_ref[:, :, _BOT, _BOT]
        o_ref[:, :, _TOP, _BOT] = x_ref[:, :, _TOP, _TOP]
        o_ref[:, :, _BOT, _TOP] = x_ref[:, :, _TOP, _BOT]
        o_ref[:, :, _BOT, _BOT] = x_ref[:, :, _BOT, _TOP]


def kernel(x):
    b, c, h, w = x.shape

    y = pl.pallas_call(
        _dma_body,
        in_specs=[pl.BlockSpec(memory_space=pl.ANY)],
        out_specs=pl.BlockSpec(memory_space=pl.ANY),
        out_shape=jax.ShapeDtypeStruct(x.shape, x.dtype),
        scratch_shapes=[pltpu.SemaphoreType.DMA((len(_PLAN),))],
    )(x)

    nblk = 48 // _CBLK

    def cmap(i, j):
        return (i, jnp.where(j < nblk, 240 // _CBLK + j, 336 // _CBLK + j - nblk), 0, 0)

    blk = pl.BlockSpec((1, _CBLK, h, w), cmap)
    return pl.pallas_call(
        _rot_body,
        grid=(b, 2 * nblk),
        in_specs=[pl.BlockSpec(memory_space=pl.ANY), blk],
        out_specs=blk,
        out_shape=jax.ShapeDtypeStruct(x.shape, x.dtype),
        input_output_aliases={0: 0},
        compiler_params=pltpu.CompilerParams(
            dimension_semantics=("parallel", "arbitrary"),
        ),
    )(y, x)


# speedup vs baseline: 10.2107x; 10.2107x over previous
import jax
import jax.numpy as jnp
from jax.experimental import pallas as pl
from jax.experimental.pallas import tpu as pltpu

_C = 384
_H = 224
_HF = 112
_CBLK = 48


def _shift_body(x_ref, o_ref):
    c = pl.program_id(1)
    g = jnp.clip((c * _CBLK - 192) // 48, 0, 3)

    @pl.when(g == 0)
    def _():
        o_ref[...] = x_ref[...]

    @pl.when(g == 1)
    def _():
        o_ref[:, :, :_HF, :_HF] = x_ref[:, :, :_HF, _HF:]
        o_ref[:, :, :_HF, _HF:] = x_ref[:, :, _HF:, :_HF]
        o_ref[:, :, _HF:, :_HF] = x_ref[:, :, _HF:, _HF:]
        o_ref[:, :, _HF:, _HF:] = x_ref[:, :, :_HF, :_HF]

    @pl.when(g == 2)
    def _():
        o_ref[:, :, :_HF, :] = x_ref[:, :, _HF:, :]
        o_ref[:, :, _HF:, :] = x_ref[:, :, :_HF, :]

    @pl.when(g == 3)
    def _():
        o_ref[:, :, :_HF, :_HF] = x_ref[:, :, _HF:, _HF:]
        o_ref[:, :, :_HF, _HF:] = x_ref[:, :, :_HF, :_HF]
        o_ref[:, :, _HF:, :_HF] = x_ref[:, :, :_HF, _HF:]
        o_ref[:, :, _HF:, _HF:] = x_ref[:, :, _HF:, :_HF]


def kernel(x):
    b, c, h, w = x.shape
    grid = (b, c // _CBLK)
    spec = pl.BlockSpec((1, _CBLK, h, w), lambda i, j: (i, j, 0, 0))
    return pl.pallas_call(
        _shift_body,
        grid=grid,
        in_specs=[spec],
        out_specs=spec,
        out_shape=jax.ShapeDtypeStruct(x.shape, x.dtype),
        compiler_params=pltpu.CompilerParams(
            dimension_semantics=("parallel", "parallel"),
        ),
    )(x)
